# baseline (speedup 1.0000x reference)
import functools

import jax
import jax.numpy as jnp
from jax import lax
from jax.experimental import pallas as pl
from jax.experimental.pallas import tpu as pltpu
from jax.experimental.pallas import tpu_sc as plsc

_NC = 2
_NS = 16
_NW = _NC * _NS
_L = 16

_P = 4
_D = 16
_A = 16
_OUT = _P * _D + _A


def _make_sc_kernel(rows, n_sym, n_cat, chunk):
    rpw = rows // _NW
    mesh = plsc.VectorSubcoreMesh(core_axis_name="c", subcore_axis_name="s")

    @functools.partial(
        pl.kernel,
        out_type=jax.ShapeDtypeStruct((rows, _OUT), jnp.float32),
        mesh=mesh,
        compiler_params=pltpu.CompilerParams(use_tc_tiling_on_sc=False),
        scratch_types=[
            pltpu.VMEM((chunk,), jnp.int32),
            pltpu.VMEM((chunk,), jnp.int32),
            pltpu.VMEM((chunk,), jnp.int32),
            pltpu.VMEM((chunk, _D), jnp.float32),
            pltpu.SemaphoreType.DMA,
        ],
    )
    def k(idx_hbm, sym_hbm, pat_hbm, aug_hbm, out_hbm,
          idx_v, addr_v, sidx_v, buf_v, sem):
        wid = lax.axis_index("s") * _NC + lax.axis_index("c")

        @pl.loop(0, rpw // chunk)
        def _chunk(ci):
            base = wid * rpw + ci * chunk
            pltpu.sync_copy(idx_hbm.at[pl.ds(base, chunk)], idx_v)
            pltpu.async_copy(aug_hbm.at[idx_v], buf_v, sem).wait()
            pltpu.sync_copy(
                buf_v, out_hbm.at[pl.ds(base, chunk), pl.ds(_P * _D, _A)])
            for p in range(_P):
                @pl.loop(0, chunk, step=_L)
                def _addr(j):
                    addr_v[pl.ds(j, _L)] = idx_v[pl.ds(j, _L)] * _P + p
                pltpu.async_copy(pat_hbm.at[addr_v], sidx_v, sem).wait()
                pltpu.async_copy(sym_hbm.at[sidx_v], buf_v, sem).wait()
                pltpu.sync_copy(
                    buf_v, out_hbm.at[pl.ds(base, chunk), pl.ds(p * _D, _D)])

    return k


def kernel(inputs, symbols, pattern, augments):
    b, f = inputs.shape
    rows = b * f
    idx_flat = inputs.reshape(rows).astype(jnp.int32)
    pattern_flat = pattern.reshape(-1).astype(jnp.int32)
    chunk = 416
    k = _make_sc_kernel(rows, symbols.shape[0], pattern.shape[0], chunk)
    out = k(idx_flat, symbols, pattern_flat, augments)
    return out.reshape(b, f, _OUT)


# speedup vs baseline: 2.8333x; 2.8333x over previous
import functools

import jax
import jax.numpy as jnp
from jax import lax
from jax.experimental import pallas as pl
from jax.experimental.pallas import tpu as pltpu
from jax.experimental.pallas import tpu_sc as plsc

_NC = 2
_NS = 16
_NW = _NC * _NS
_L = 16

_P = 4
_D = 16
_A = 16
_OUT = _P * _D + _A


def _make_sc_kernel(rows, n_sym, n_cat, chunk):
    rpw = rows // _NW
    mesh = plsc.VectorSubcoreMesh(core_axis_name="c", subcore_axis_name="s")

    @functools.partial(
        pl.kernel,
        out_type=jax.ShapeDtypeStruct((rows, _OUT), jnp.float32),
        mesh=mesh,
        compiler_params=pltpu.CompilerParams(use_tc_tiling_on_sc=False),
        scratch_types=[
            pltpu.VMEM((chunk,), jnp.int32),
            pltpu.VMEM((chunk,), jnp.int32),
            pltpu.VMEM((chunk,), jnp.int32),
            pltpu.VMEM((chunk, _D), jnp.float32),
            pltpu.SemaphoreType.DMA,
        ],
    )
    def k(idx_hbm, sym_hbm, pat_hbm, aug_hbm, out_hbm,
          idx_v, addr_v, sidx_v, buf_v, sem):
        wid = lax.axis_index("s") * _NC + lax.axis_index("c")

        @pl.loop(0, rpw // chunk)
        def _chunk(ci):
            base = wid * rpw + ci * chunk
            pltpu.sync_copy(idx_hbm.at[pl.ds(base, chunk)], idx_v)
            pltpu.async_copy(aug_hbm.at[idx_v], buf_v, sem).wait()
            pltpu.sync_copy(
                buf_v, out_hbm.at[pl.ds(base, chunk), pl.ds(_P * _D, _A)])
            for p in range(_P):
                @pl.loop(0, chunk, step=_L)
                def _addr(j):
                    addr_v[pl.ds(j, _L)] = idx_v[pl.ds(j, _L)] + p * n_cat
                pltpu.async_copy(pat_hbm.at[addr_v], sidx_v, sem).wait()
                pltpu.async_copy(sym_hbm.at[sidx_v], buf_v, sem).wait()
                pltpu.sync_copy(
                    buf_v, out_hbm.at[pl.ds(base, chunk), pl.ds(p * _D, _D)])

    return k


def kernel(inputs, symbols, pattern, augments):
    b, f = inputs.shape
    rows = b * f
    idx_flat = inputs.reshape(rows).astype(jnp.int32)
    pattern_flat = pattern.T.reshape(-1).astype(jnp.int32)
    chunk = 416
    k = _make_sc_kernel(rows, symbols.shape[0], pattern.shape[0], chunk)
    out = k(idx_flat, symbols, pattern_flat, augments)
    return out.reshape(b, f, _OUT)
